# baseline (speedup 1.0000x reference)
import jax
import jax.numpy as jnp
from jax import lax
from jax.experimental import pallas as pl
from jax.experimental.pallas import tpu as pltpu
from jax.experimental.pallas import tpu_sc as plsc

NC = 2
NS = 16
L = 16
NW = NC * NS

BATCH = 16384
EMBED_K = 16
B_PER_W = BATCH // NW
CHUNK = 128
N_CHUNKS = B_PER_W // CHUNK
N_GROUPS = B_PER_W // L


def _sc_body(x_hbm, w_hbm, h_hbm, out_hbm,
             xv, uidx_v, vidx_v, urows, vrows, out_v, sem):
    wid = lax.axis_index("s") * NC + lax.axis_index("c")
    base = wid * B_PER_W

    pltpu.sync_copy(x_hbm.at[pl.ds(base, B_PER_W)], xv)

    iota = lax.iota(jnp.int32, L)
    zeros = jnp.zeros((L,), jnp.int32)
    ones = jnp.ones((L,), jnp.int32)
    for g in range(N_GROUPS):
        row = g * L + iota
        c, o = divmod(g * L, CHUNK)
        uidx_v[c, pl.ds(o, L)] = plsc.load_gather(xv, [row, zeros])
        vidx_v[c, pl.ds(o, L)] = plsc.load_gather(xv, [row, ones])

    descs = []
    for c in range(N_CHUNKS):
        dst = pl.ds(c * CHUNK, CHUNK)
        descs.append(pltpu.async_copy(w_hbm.at[uidx_v.at[c]], urows.at[dst], sem))
        descs.append(pltpu.async_copy(h_hbm.at[vidx_v.at[c]], vrows.at[dst], sem))
    for d in descs:
        d.wait()

    def group(g, _):
        row = g * L + iota
        acc = jnp.zeros((L,), jnp.float32)
        for j in range(EMBED_K):
            col = jnp.full((L,), j, jnp.int32)
            u = plsc.load_gather(urows, [row, col])
            v = plsc.load_gather(vrows, [row, col])
            acc = acc + u * v
        out_v[pl.ds(g * L, L)] = 1.0 / (1.0 + jnp.exp(-acc))
        return _

    lax.fori_loop(0, N_GROUPS, group, None)

    pltpu.sync_copy(out_v, out_hbm.at[pl.ds(base, B_PER_W)])


@jax.jit
def kernel(x, W, H):
    mesh = plsc.VectorSubcoreMesh(core_axis_name="c", subcore_axis_name="s")
    run = pl.kernel(
        _sc_body,
        out_type=jax.ShapeDtypeStruct((BATCH,), jnp.float32),
        mesh=mesh,
        compiler_params=pltpu.CompilerParams(
            use_tc_tiling_on_sc=False, needs_layout_passes=False),
        scratch_types=[
            pltpu.VMEM((B_PER_W, 2), jnp.int32),
            pltpu.VMEM((N_CHUNKS, CHUNK), jnp.int32),
            pltpu.VMEM((N_CHUNKS, CHUNK), jnp.int32),
            pltpu.VMEM((B_PER_W, EMBED_K), jnp.float32),
            pltpu.VMEM((B_PER_W, EMBED_K), jnp.float32),
            pltpu.VMEM((B_PER_W,), jnp.float32),
            pltpu.SemaphoreType.DMA,
        ],
    )
    return run(x, W, H)


# speedup vs baseline: 1.3817x; 1.3817x over previous
import jax
import jax.numpy as jnp
from jax import lax
from jax.experimental import pallas as pl
from jax.experimental.pallas import tpu as pltpu
from jax.experimental.pallas import tpu_sc as plsc

NC = 2
NS = 16
L = 16
NW = NC * NS

BATCH = 16384
EMBED_K = 16
B_PER_W = BATCH // NW
N_GROUPS = B_PER_W // L
N_PAIRS = N_GROUPS // 2


def _sc_body(uix_hbm, vix_hbm, w_hbm, h_hbm, out_hbm,
             uidx_v, vidx_v, wblk_a, hblk_a, wblk_b, hblk_b, out_v,
             sem_a, sem_b):
    wid = lax.axis_index("s") * NC + lax.axis_index("c")
    base = wid * B_PER_W

    pltpu.sync_copy(uix_hbm.at[pl.ds(base, B_PER_W)], uidx_v)
    pltpu.sync_copy(vix_hbm.at[pl.ds(base, B_PER_W)], vidx_v)

    iota = lax.iota(jnp.int32, L)
    eights = jnp.full((L,), 8, jnp.int32)

    def fire(g, wblk, hblk, sem):
        uvec = uidx_v[pl.ds(g * L, L)]
        vvec = vidx_v[pl.ds(g * L, L)]
        for l in range(L):
            ub = pl.multiple_of((uvec[l] // 8) * 8, 8)
            vb = pl.multiple_of((vvec[l] // 8) * 8, 8)
            pltpu.async_copy(w_hbm.at[pl.ds(ub, 8), :], wblk.at[l], sem)
            pltpu.async_copy(h_hbm.at[pl.ds(vb, 8), :], hblk.at[l], sem)

    def drain(wblk, hblk, sem):
        for l in range(L):
            pltpu.make_async_copy(w_hbm.at[pl.ds(0, 8), :], wblk.at[l], sem).wait()
            pltpu.make_async_copy(h_hbm.at[pl.ds(0, 8), :], hblk.at[l], sem).wait()

    def compute(g, wblk, hblk):
        uvec = uidx_v[pl.ds(g * L, L)]
        vvec = vidx_v[pl.ds(g * L, L)]
        rw = lax.rem(uvec, eights)
        rh = lax.rem(vvec, eights)
        acc = jnp.zeros((L,), jnp.float32)
        for k in range(EMBED_K):
            col = jnp.full((L,), k, jnp.int32)
            wk = plsc.load_gather(wblk, [iota, rw, col])
            hk = plsc.load_gather(hblk, [iota, rh, col])
            acc = acc + wk * hk
        out_v[pl.ds(g * L, L)] = 1.0 / (1.0 + jnp.exp(-acc))

    fire(0, wblk_a, hblk_a, sem_a)

    def pair(i, _):
        ga = 2 * i
        gb = 2 * i + 1
        fire(gb, wblk_b, hblk_b, sem_b)
        drain(wblk_a, hblk_a, sem_a)
        compute(ga, wblk_a, hblk_a)

        @pl.when(i < N_PAIRS - 1)
        def _():
            fire(gb + 1, wblk_a, hblk_a, sem_a)

        drain(wblk_b, hblk_b, sem_b)
        compute(gb, wblk_b, hblk_b)
        return _

    lax.fori_loop(0, N_PAIRS, pair, None)

    pltpu.sync_copy(out_v, out_hbm.at[pl.ds(base, B_PER_W)])


@jax.jit
def kernel(x, W, H):
    u_idx = x[:, 0].astype(jnp.int32)
    v_idx = x[:, 1].astype(jnp.int32)
    mesh = plsc.VectorSubcoreMesh(core_axis_name="c", subcore_axis_name="s")
    blk = pltpu.VMEM((L, 8, EMBED_K), jnp.float32)
    run = pl.kernel(
        _sc_body,
        out_type=jax.ShapeDtypeStruct((BATCH,), jnp.float32),
        mesh=mesh,
        compiler_params=pltpu.CompilerParams(
            use_tc_tiling_on_sc=True, needs_layout_passes=False),
        scratch_types=[
            pltpu.VMEM((B_PER_W,), jnp.int32),
            pltpu.VMEM((B_PER_W,), jnp.int32),
            blk, blk, blk, blk,
            pltpu.VMEM((B_PER_W,), jnp.float32),
            pltpu.SemaphoreType.DMA,
            pltpu.SemaphoreType.DMA,
        ],
    )
    return run(u_idx, v_idx, W, H)
